# Initial kernel scaffold; baseline (speedup 1.0000x reference)
import jax
import jax.numpy as jnp
from jax.experimental import pallas as pl


def kernel(corr, kc, problem, A, trans_logits, obs_logits_problem, init_logits):
    raise NotImplementedError("write your pallas kernel here")


# speedup vs baseline: 2.0620x; 2.0620x over previous
import functools

import jax
import jax.numpy as jnp
from jax import lax
from jax.experimental import pallas as pl
from jax.experimental.pallas import tpu as pltpu

_NP = 100000
_K = 64
_B, _T = 1024, 50
_N = _B * _T
_BB = 128


def _make_sc_gather():
    from jax.experimental.pallas import tpu_sc as plsc

    info = plsc.get_sparse_core_info()
    nc, ns = info.num_cores, info.num_subcores
    nw = nc * ns
    per_w = _N // nw
    nch = 2
    ch = per_w // nch

    mesh = plsc.VectorSubcoreMesh(core_axis_name="c", subcore_axis_name="s")

    @functools.partial(
        pl.kernel,
        mesh=mesh,
        compiler_params=pltpu.CompilerParams(use_tc_tiling_on_sc=False),
        out_type=[
            jax.ShapeDtypeStruct((_N, _K), jnp.float32),
            jax.ShapeDtypeStruct((_N, 16), jnp.float32),
        ],
        scratch_types=[
            pltpu.VMEM((nch, ch), jnp.int32),
            pltpu.VMEM((ch, _K), jnp.float32),
            pltpu.VMEM((nch, ch), jnp.int32),
            pltpu.VMEM((ch, 16), jnp.float32),
            pltpu.SemaphoreType.DMA,
        ],
    )
    def gather(a_hbm, kc_hbm, obs_hbm, pr_hbm, chain_out, obs_out,
               kidx_v, rows_v, pidx_v, orows_v, sem):
        wid = lax.axis_index("s") * nc + lax.axis_index("c")
        base = wid * per_w
        for h in range(nch):
            pltpu.sync_copy(kc_hbm.at[pl.ds(base + h * ch, ch)], kidx_v.at[h])
            pltpu.sync_copy(pr_hbm.at[pl.ds(base + h * ch, ch)], pidx_v.at[h])
        for h in range(nch):
            c1 = pltpu.async_copy(a_hbm.at[kidx_v.at[h]], rows_v, sem)
            c2 = pltpu.async_copy(obs_hbm.at[pidx_v.at[h]], orows_v, sem)
            c1.wait()
            c2.wait()
            pltpu.sync_copy(rows_v, chain_out.at[pl.ds(base + h * ch, ch)])
            pltpu.sync_copy(orows_v, obs_out.at[pl.ds(base + h * ch, ch)])

    return gather


def _scan_body(chain_ref, obs_ref, corr_ref, rr_ref, lt_ref, il_ref, out_ref):
    lt0 = lt_ref[0:1, :]
    lt1 = lt_ref[1:2, :]
    lt2 = lt_ref[2:3, :]
    lt3 = lt_ref[3:4, :]
    d0 = jnp.logaddexp(lt0, lt2)
    d1 = jnp.logaddexp(lt1, lt3)
    lt00 = lt0 - d0
    lt01 = lt1 - d1
    lt10 = lt2 - d0
    lt11 = lt3 - d1
    i0 = il_ref[0:1, :]
    i1 = il_ref[1:2, :]
    zi = jnp.logaddexp(i0, i1)
    al0 = jnp.broadcast_to(i0 - zi, (_BB, _K))
    al1 = jnp.broadcast_to(i1 - zi, (_BB, _K))

    def step(t, carry):
        al0, al1 = carry
        c = chain_ref[t]
        ob = obs_ref[t]
        y = corr_ref[t]
        r = rr_ref[t]
        lo = jnp.where(
            r < 2,
            jnp.where(r == 0, ob[:, 0:4], ob[:, 4:8]),
            jnp.where(r == 2, ob[:, 8:12], ob[:, 12:16]),
        )
        lse0 = jnp.logaddexp(lo[:, 0:1], lo[:, 1:2])
        lse1 = jnp.logaddexp(lo[:, 2:3], lo[:, 3:4])
        lb0 = lo[:, 0:1] - lse0
        lb1 = lo[:, 1:2] - lse0
        lb2 = lo[:, 2:3] - lse1
        lb3 = lo[:, 3:4] - lse1
        s = jnp.sum(c, axis=1, keepdims=True)
        a20 = jnp.sum(c * al0, axis=1, keepdims=True)
        a21 = jnp.sum(c * al1, axis=1, keepdims=True)
        t00 = s * lb0 + a20
        t01 = s * lb1 + a20
        t10 = s * lb2 + a21
        t11 = s * lb3 + a21
        lp0 = jnp.logaddexp(t00, t10)
        lp1 = jnp.logaddexp(t01, t11)
        z = jnp.logaddexp(lp0, lp1)
        out_ref[t] = jnp.concatenate([lp0 - z, lp1 - z], axis=1)
        ay0 = jnp.where(y == 0, lb0, lb1)
        ay1 = jnp.where(y == 0, lb2, lb3)
        m00 = jnp.sum(c * lt00, axis=1, keepdims=True)
        m01 = jnp.sum(c * lt01, axis=1, keepdims=True)
        m10 = jnp.sum(c * lt10, axis=1, keepdims=True)
        m11 = jnp.sum(c * lt11, axis=1, keepdims=True)
        u00 = s * ay0 + a20 + m00
        u01 = s * ay1 + a21 + m01
        u10 = s * ay0 + a20 + m10
        u11 = s * ay1 + a21 + m11
        a3_0 = jnp.logaddexp(u00, u01)
        a3_1 = jnp.logaddexp(u10, u11)
        al0 = (1.0 - c) * al0 + c * a3_0
        al1 = (1.0 - c) * al1 + c * a3_1
        return al0, al1

    lax.fori_loop(0, _T, step, (al0, al1))


def _scan_call(chain3, obs3, corr3, rr3, lt4, il2, interpret=False):
    return pl.pallas_call(
        _scan_body,
        grid=(_B // _BB,),
        in_specs=[
            pl.BlockSpec((_T, _BB, _K), lambda i: (0, i, 0)),
            pl.BlockSpec((_T, _BB, 16), lambda i: (0, i, 0)),
            pl.BlockSpec((_T, _BB, 1), lambda i: (0, i, 0)),
            pl.BlockSpec((_T, _BB, 1), lambda i: (0, i, 0)),
            pl.BlockSpec((4, _K), lambda i: (0, 0)),
            pl.BlockSpec((2, _K), lambda i: (0, 0)),
        ],
        out_specs=pl.BlockSpec((_T, _BB, 2), lambda i: (0, i, 0)),
        out_shape=jax.ShapeDtypeStruct((_T, _B, 2), jnp.float32),
        interpret=interpret,
    )(chain3, obs3, corr3, rr3, lt4, il2)


def kernel(corr, kc, problem, A, trans_logits, obs_logits_problem, init_logits):
    kc_t = kc.T.reshape(_N).astype(jnp.int32)
    pr_t = problem.T.reshape(_N).astype(jnp.int32)
    obs16 = obs_logits_problem.reshape(_NP // 4, 16)
    chain_flat, obs_g = _make_sc_gather()(A, kc_t, obs16, pr_t // 4)
    chain3 = chain_flat.reshape(_T, _B, _K)
    obs3 = obs_g.reshape(_T, _B, 16)
    corr3 = corr.T.reshape(_T, _B, 1).astype(jnp.int32)
    rr3 = (pr_t % 4).reshape(_T, _B, 1)
    lt4 = jnp.transpose(trans_logits, (1, 2, 0)).reshape(4, _K)
    il2 = init_logits.T
    outs = _scan_call(chain3, obs3, corr3, rr3, lt4, il2)
    return jnp.transpose(outs, (1, 0, 2))
